# baseline (speedup 1.0000x reference)
import functools

import jax
import jax.numpy as jnp
import numpy as np
from jax import lax
from jax.experimental import pallas as pl
from jax.experimental.pallas import tpu as pltpu

_TAU = 1.0
_TINY = float(np.finfo(np.float32).tiny)
_BLOCK_ROWS = 8


def _rotl(v, r):
    return lax.shift_left(v, jnp.uint32(r)) | lax.shift_right_logical(
        v, jnp.uint32(32 - r))


def _threefry_bits(j):
    ks0 = jnp.uint32(0)
    ks1 = jnp.uint32(42)
    ks2 = jnp.uint32(0 ^ 42 ^ 0x1BD11BDA)
    rot0 = (13, 15, 26, 6)
    rot1 = (17, 29, 16, 24)

    x0 = jnp.full(j.shape, ks0, jnp.uint32)
    x1 = j + ks1

    def rounds(x0, x1, rots):
        for r in rots:
            x0 = x0 + x1
            x1 = x0 ^ _rotl(x1, r)
        return x0, x1

    x0, x1 = rounds(x0, x1, rot0)
    x0, x1 = x0 + ks1, x1 + ks2 + jnp.uint32(1)
    x0, x1 = rounds(x0, x1, rot1)
    x0, x1 = x0 + ks2, x1 + ks0 + jnp.uint32(2)
    x0, x1 = rounds(x0, x1, rot0)
    x0, x1 = x0 + ks0, x1 + ks1 + jnp.uint32(3)
    x0, x1 = rounds(x0, x1, rot1)
    x0, x1 = x0 + ks1, x1 + ks2 + jnp.uint32(4)
    x0, x1 = rounds(x0, x1, rot0)
    x0, x1 = x0 + ks2, x1 + ks0 + jnp.uint32(5)
    return x0 ^ x1


def _body(x_ref, o_ref, *, block_rows, n_cols):
    pid = pl.program_id(0)
    xb = x_ref[...] * _TAU

    m = jnp.max(xb, axis=1, keepdims=True)
    e = jnp.exp(xb - m)
    s = jnp.sum(e, axis=1, keepdims=True)
    logits = jnp.log(e / s + 1e-20)

    row = lax.broadcasted_iota(jnp.uint32, xb.shape, 0)
    col = lax.broadcasted_iota(jnp.uint32, xb.shape, 1)
    base = pid.astype(jnp.uint32) * jnp.uint32(block_rows * n_cols)
    j = base + row * jnp.uint32(n_cols) + col

    bits = _threefry_bits(j)
    fb = lax.bitcast_convert_type(
        lax.shift_right_logical(bits, jnp.uint32(9)) | jnp.uint32(0x3F800000),
        jnp.float32) - jnp.float32(1.0)
    u = jnp.maximum(jnp.float32(_TINY), fb + jnp.float32(_TINY))
    g = -jnp.log(-jnp.log(u))

    score = logits + g
    rowmax = jnp.max(score, axis=1, keepdims=True)
    colidx = lax.broadcasted_iota(jnp.int32, xb.shape, 1)
    idx = jnp.min(jnp.where(score == rowmax, colidx, jnp.int32(n_cols)),
                  axis=1, keepdims=True)
    o_ref[...] = (colidx == idx).astype(o_ref.dtype)


@jax.jit
def kernel(x):
    n_rows, n_cols = x.shape
    block_rows = _BLOCK_ROWS
    grid = n_rows // block_rows
    return pl.pallas_call(
        functools.partial(_body, block_rows=block_rows, n_cols=n_cols),
        out_shape=jax.ShapeDtypeStruct(x.shape, x.dtype),
        grid=(grid,),
        in_specs=[pl.BlockSpec((block_rows, n_cols), lambda i: (i, 0))],
        out_specs=pl.BlockSpec((block_rows, n_cols), lambda i: (i, 0)),
        compiler_params=pltpu.CompilerParams(
            dimension_semantics=("parallel",)),
    )(x)
